# baseline (speedup 1.0000x reference)
import jax
import jax.numpy as jnp
from jax.experimental import pallas as pl
from jax.experimental.pallas import tpu as pltpu

N, L, E = 2, 1024, 256
H, P, K = 8, 4, 4
D = E // H
NH = N * H
LB = 256
NB = L // LB
OA = P * 3 + P
INF = 3.0e38


def _fused_kernel(qf_ref, wc_ref, bc_ref, rpq_ref, rpkT_ref, wout_ref,
                  bout_ref, o_ref, valh, oah):
    j = pl.program_id(0)
    i = pl.program_id(1)

    @pl.when(jnp.logical_and(j == 0, i == 0))
    def _proj():
        x = (jnp.dot(qf_ref[...], wc_ref[...],
                     preferred_element_type=jnp.float32)
             + bc_ref[...])
        for h in range(H):
            valh[h] = x[:, h * D:(h + 1) * D]
            oah[h, :, 0:P * 3] = x[:, E + h * P * 3:E + (h + 1) * P * 3]
            oah[h, :, P * 3:OA] = x[:, E + H * P * 3 + h * P:
                                    E + H * P * 3 + (h + 1) * P]

    M = P * LB
    rpkT = rpkT_ref[0]
    rpk2 = (rpkT[0:1, :] * rpkT[0:1, :]
            + rpkT[1:2, :] * rpkT[1:2, :]
            + rpkT[2:3, :] * rpkT[2:3, :])

    oa = oah[i % H, pl.ds((i // H) * L + j * LB, LB)]
    val = valh[i // N, pl.ds((i % N) * L, L)]

    a_cols = [oa[:, P * 3 + p:P * 3 + p + 1] for p in range(P)]
    amax = jnp.maximum(jnp.maximum(a_cols[0], a_cols[1]),
                       jnp.maximum(a_cols[2], a_cols[3]))
    e_cols = [jnp.exp(a - amax) for a in a_cols]
    esum = e_cols[0] + e_cols[1] + e_cols[2] + e_cols[3]
    att4 = jnp.concatenate([e / esum for e in e_cols], axis=0)

    rpq = rpq_ref[0]
    samp = jnp.concatenate(
        [rpq + oa[:, 3 * p:3 * p + 3] for p in range(P)], axis=0)
    s2 = (samp[:, 0:1] * samp[:, 0:1]
          + samp[:, 1:2] * samp[:, 1:2]
          + samp[:, 2:3] * samp[:, 2:3])
    cross = jax.lax.dot_general(
        samp.astype(jnp.bfloat16), rpkT.astype(jnp.bfloat16),
        (((1,), (0,)), ((), ())),
        preferred_element_type=jnp.float32)
    d2 = jnp.maximum(s2 + rpk2 - 2.0 * cross, 0.0)

    fi = jax.lax.broadcasted_iota(jnp.int32, (M, L), 1).astype(jnp.float32)
    acc = jnp.zeros((M, L), jnp.float32)
    ssum = jnp.zeros((M, 1), jnp.float32)
    for k in range(K):
        rowmin = jnp.min(d2, axis=-1, keepdims=True)
        cand = jnp.where(d2 == rowmin, fi, jnp.float32(L))
        idx = jnp.min(cand, axis=-1, keepdims=True)
        sel = cand == idx
        w = 1.0 / (jnp.sqrt(rowmin) + 1e-8)
        acc = jnp.where(sel, w, acc)
        ssum = ssum + w
        if k + 1 < K:
            d2 = jnp.where(sel, INF, d2)
    wmat = acc * (att4 / ssum)

    out4 = jnp.dot(wmat, val, preferred_element_type=jnp.float32)
    head = (out4[0 * LB:1 * LB] + out4[1 * LB:2 * LB]
            + out4[2 * LB:3 * LB] + out4[3 * LB:4 * LB])

    @pl.when(i % H == 0)
    def _init():
        o_ref[0] = jnp.broadcast_to(bout_ref[...], (LB, E))

    o_ref[0] += jnp.dot(head, wout_ref[...],
                        preferred_element_type=jnp.float32)


def kernel(query_features, reference_points, W_val, b_val, W_off, b_off,
           W_att, b_att, W_out, b_out):
    qf = query_features.reshape(N * L, E)
    Wc = jnp.concatenate([W_val, W_off, W_att], axis=1)
    bc = jnp.concatenate([b_val, b_off, b_att]).reshape(1, -1)
    rpT = reference_points.transpose(0, 2, 1)

    out = pl.pallas_call(
        _fused_kernel,
        grid=(NB, NH),
        in_specs=[
            pl.BlockSpec((N * L, E), lambda j, i: (0, 0)),
            pl.BlockSpec((E, E + H * P * 3 + H * P), lambda j, i: (0, 0)),
            pl.BlockSpec((1, E + H * P * 3 + H * P), lambda j, i: (0, 0)),
            pl.BlockSpec((1, LB, 3), lambda j, i: (i // H, j, 0)),
            pl.BlockSpec((1, 3, L), lambda j, i: (i % N, 0, 0)),
            pl.BlockSpec((D, E), lambda j, i: (i % H, 0)),
            pl.BlockSpec((1, E), lambda j, i: (0, 0)),
        ],
        out_specs=pl.BlockSpec((1, LB, E), lambda j, i: (i // H, j, 0)),
        out_shape=jax.ShapeDtypeStruct((N, L, E), jnp.float32),
        scratch_shapes=[
            pltpu.VMEM((H, N * L, D), jnp.float32),
            pltpu.VMEM((H, N * L, OA), jnp.float32),
        ],
    )(qf, Wc, bc, reference_points, rpT, W_out, b_out.reshape(1, E))
    return out


# speedup vs baseline: 1.0234x; 1.0234x over previous
import jax
import jax.numpy as jnp
from jax.experimental import pallas as pl
from jax.experimental.pallas import tpu as pltpu

N, L, E = 2, 1024, 256
H, P, K = 8, 4, 4
D = E // H
NH = N * H
LB = 512
NB = L // LB
OA = P * 3 + P
INF = 3.0e38


def _fused_kernel(qf_ref, wc_ref, bc_ref, rpq_ref, rpkT_ref, wout_ref,
                  bout_ref, o_ref, valh, oah):
    j = pl.program_id(0)
    i = pl.program_id(1)

    @pl.when(jnp.logical_and(j == 0, i == 0))
    def _proj():
        x = (jnp.dot(qf_ref[...], wc_ref[...],
                     preferred_element_type=jnp.float32)
             + bc_ref[...])
        for h in range(H):
            valh[h] = x[:, h * D:(h + 1) * D]
            oah[h, :, 0:P * 3] = x[:, E + h * P * 3:E + (h + 1) * P * 3]
            oah[h, :, P * 3:OA] = x[:, E + H * P * 3 + h * P:
                                    E + H * P * 3 + (h + 1) * P]

    M = P * LB
    rpkT = rpkT_ref[0]
    rpk2 = (rpkT[0:1, :] * rpkT[0:1, :]
            + rpkT[1:2, :] * rpkT[1:2, :]
            + rpkT[2:3, :] * rpkT[2:3, :])

    oa = oah[i % H, pl.ds((i // H) * L + j * LB, LB)]
    val = valh[i // N, pl.ds((i % N) * L, L)]

    a_cols = [oa[:, P * 3 + p:P * 3 + p + 1] for p in range(P)]
    amax = jnp.maximum(jnp.maximum(a_cols[0], a_cols[1]),
                       jnp.maximum(a_cols[2], a_cols[3]))
    e_cols = [jnp.exp(a - amax) for a in a_cols]
    esum = e_cols[0] + e_cols[1] + e_cols[2] + e_cols[3]
    att4 = jnp.concatenate([e / esum for e in e_cols], axis=0)

    rpq = rpq_ref[0]
    samp = jnp.concatenate(
        [rpq + oa[:, 3 * p:3 * p + 3] for p in range(P)], axis=0)
    s2 = (samp[:, 0:1] * samp[:, 0:1]
          + samp[:, 1:2] * samp[:, 1:2]
          + samp[:, 2:3] * samp[:, 2:3])
    cross = jax.lax.dot_general(
        samp.astype(jnp.bfloat16), rpkT.astype(jnp.bfloat16),
        (((1,), (0,)), ((), ())),
        preferred_element_type=jnp.float32)
    d2 = jnp.maximum(s2 + rpk2 - 2.0 * cross, 0.0)

    fi = jax.lax.broadcasted_iota(jnp.int32, (M, L), 1).astype(jnp.float32)
    idxs, ws = [], []
    ssum = jnp.zeros((M, 1), jnp.float32)
    for k in range(K):
        rowmin = jnp.min(d2, axis=-1, keepdims=True)
        cand = jnp.where(d2 == rowmin, fi, jnp.float32(L))
        idx = jnp.min(cand, axis=-1, keepdims=True)
        w = 1.0 / (jnp.sqrt(rowmin) + 1e-8)
        idxs.append(idx)
        ws.append(w)
        ssum = ssum + w
        if k + 1 < K:
            d2 = jnp.where(cand == idx, INF, d2)
    t = att4 / ssum
    wmat = jnp.zeros((M, L), jnp.float32)
    for k in range(K):
        wmat = jnp.where(fi == idxs[k], ws[k] * t, wmat)

    out4 = jnp.dot(wmat, val, preferred_element_type=jnp.float32)
    head = (out4[0 * LB:1 * LB] + out4[1 * LB:2 * LB]
            + out4[2 * LB:3 * LB] + out4[3 * LB:4 * LB])

    @pl.when(i % H == 0)
    def _init():
        o_ref[0] = jnp.broadcast_to(bout_ref[...], (LB, E))

    o_ref[0] += jnp.dot(head, wout_ref[...],
                        preferred_element_type=jnp.float32)


def kernel(query_features, reference_points, W_val, b_val, W_off, b_off,
           W_att, b_att, W_out, b_out):
    qf = query_features.reshape(N * L, E)
    Wc = jnp.concatenate([W_val, W_off, W_att], axis=1)
    bc = jnp.concatenate([b_val, b_off, b_att]).reshape(1, -1)
    rpT = reference_points.transpose(0, 2, 1)

    out = pl.pallas_call(
        _fused_kernel,
        grid=(NB, NH),
        in_specs=[
            pl.BlockSpec((N * L, E), lambda j, i: (0, 0)),
            pl.BlockSpec((E, E + H * P * 3 + H * P), lambda j, i: (0, 0)),
            pl.BlockSpec((1, E + H * P * 3 + H * P), lambda j, i: (0, 0)),
            pl.BlockSpec((1, LB, 3), lambda j, i: (i // H, j, 0)),
            pl.BlockSpec((1, 3, L), lambda j, i: (i % N, 0, 0)),
            pl.BlockSpec((D, E), lambda j, i: (i % H, 0)),
            pl.BlockSpec((1, E), lambda j, i: (0, 0)),
        ],
        out_specs=pl.BlockSpec((1, LB, E), lambda j, i: (i // H, j, 0)),
        out_shape=jax.ShapeDtypeStruct((N, L, E), jnp.float32),
        scratch_shapes=[
            pltpu.VMEM((H, N * L, D), jnp.float32),
            pltpu.VMEM((H, N * L, OA), jnp.float32),
        ],
    )(qf, Wc, bc, reference_points, rpT, W_out, b_out.reshape(1, E))
    return out
